# baseline (speedup 1.0000x reference)
import jax
import jax.numpy as jnp
from jax import lax
from jax.experimental import pallas as pl
from jax.experimental.pallas import tpu as pltpu
from jax.experimental.pallas import tpu_sc as plsc

N = 10000
E = 320000
D = 128
D3 = 384
EPS = 1e-5
NC = 2
NS = 16
NW = NC * NS
EW = E // NW
RPA = 624

_MESH = plsc.VectorSubcoreMesh(
    core_axis_name="c", subcore_axis_name="s", num_cores=NC, num_subcores=NS)


def _worker():
    cid = lax.axis_index("c")
    sid = lax.axis_index("s")
    return cid, sid, sid * NC + cid


_C0 = 64
_NF0 = EW // _C0 - 1
_T0 = 16


def _counts_body(i0_h, i1_h, i2_h, out_h, c_s,
                 ib0, it0, ones_v, zb_v):
    cid, sid, wid = _worker()
    row0 = sid * RPA

    def zrow(r, _):
        for j8 in range(8):
            zb_v[r, pl.ds(j8 * 16, 16)] = jnp.zeros((16,), jnp.float32)
            ones_v[r, pl.ds(j8 * 16, 16)] = jnp.full((16,), 1.0, jnp.float32)
        return 0
    lax.fori_loop(0, _C0, zrow, 0)

    base0 = wid * EW
    for k, i_h in enumerate((i0_h, i1_h, i2_h)):
        def zchunk(t, _):
            pltpu.sync_copy(zb_v, c_s.at[pl.ds(row0 + t * _C0, _C0)])
            return 0
        lax.fori_loop(0, 9, zchunk, 0)
        pltpu.sync_copy(zb_v.at[pl.ds(0, 48)],
                        c_s.at[pl.ds(row0 + 9 * _C0, 48)])

        @pl.when(sid == NS - 1)
        def _():
            pltpu.sync_copy(zb_v.at[pl.ds(0, 16)],
                            c_s.at[pl.ds(NS * RPA, 16)])
        plsc.subcore_barrier()

        def chunk(t, _):
            b = base0 + t * _C0
            pltpu.sync_copy(i_h.at[pl.ds(b, _C0)], ib0)
            pltpu.sync_copy(ones_v, c_s.at[ib0], add=True)
            return 0
        lax.fori_loop(0, _NF0 + 1, chunk, 0)
        bt = base0 + (_NF0 + 1) * _C0
        pltpu.sync_copy(i_h.at[pl.ds(bt, _T0)], it0)
        pltpu.sync_copy(ones_v.at[pl.ds(0, _T0)], c_s.at[it0], add=True)
        plsc.subcore_barrier()

        pltpu.sync_copy(c_s.at[pl.ds(row0, RPA)],
                        out_h.at[cid, k, pl.ds(row0, RPA)])

        @pl.when(sid == NS - 1)
        def _():
            pltpu.sync_copy(c_s.at[pl.ds(NS * RPA, 16)],
                            out_h.at[cid, k, pl.ds(NS * RPA, 16)])
        plsc.subcore_barrier()


_counts_call = pl.kernel(
    _counts_body,
    out_type=jax.ShapeDtypeStruct((NC, 3, N, D), jnp.float32),
    mesh=_MESH,
    scratch_types=dict(
        c_s=pltpu.VMEM_SHARED((N, D), jnp.float32),
        ib0=pltpu.VMEM((_C0,), jnp.int32),
        it0=pltpu.VMEM((_T0,), jnp.int32),
        ones_v=pltpu.VMEM((_C0, D), jnp.float32),
        zb_v=pltpu.VMEM((_C0, D), jnp.float32),
    ),
    name="sc_counts",
)


_NB1 = 5
_BR = N // _NB1


def _k1_body(x_ref, wt_ref, wm_ref, wb_ref, cp_ref,
             q0_ref, q1_ref, q2_ref, v0_ref, v1_ref, v2_ref, sz_ref, acc_ref):
    i = pl.program_id(0)
    x = x_ref[...]
    cp = cp_ref[...]

    @pl.when(i == 0)
    def _():
        acc_ref[...] = jnp.zeros_like(acc_ref)

    acc = acc_ref[...]
    for j, (q_ref, v_ref) in enumerate(
            ((q0_ref, v0_ref), (q1_ref, v1_ref), (q2_ref, v2_ref))):
        parts = []
        for w_ref in (wt_ref, wm_ref, wb_ref):
            ws = w_ref[:, j * D:(j + 1) * D]
            parts.append(lax.dot_general(
                x, ws, (((1,), (1,)), ((), ())),
                preferred_element_type=jnp.float32))
        q = jnp.concatenate(parts, axis=1)
        q_ref[...] = q
        cnt = (cp[0, j, :, 0] + cp[1, j, :, 0])[:, None]
        v_ref[...] = jnp.broadcast_to(
            1.0 / jnp.maximum(cnt, 1.0), (_BR, D))
        acc = acc + jnp.sum(q * cnt, axis=0, keepdims=True)
    acc_ref[...] = acc

    @pl.when(i == _NB1 - 1)
    def _():
        sz_ref[...] = acc_ref[...]


def _k1_call(x, wt, wm, wb, cnt_part):
    return pl.pallas_call(
        _k1_body,
        grid=(_NB1,),
        in_specs=[
            pl.BlockSpec((_BR, D), lambda i: (i, 0)),
            pl.BlockSpec((D, D3), lambda i: (0, 0)),
            pl.BlockSpec((D, D3), lambda i: (0, 0)),
            pl.BlockSpec((D, D3), lambda i: (0, 0)),
            pl.BlockSpec((NC, 3, _BR, D), lambda i: (0, 0, i, 0)),
        ],
        out_specs=[
            pl.BlockSpec((_BR, D3), lambda i: (i, 0)),
            pl.BlockSpec((_BR, D3), lambda i: (i, 0)),
            pl.BlockSpec((_BR, D3), lambda i: (i, 0)),
            pl.BlockSpec((_BR, D), lambda i: (i, 0)),
            pl.BlockSpec((_BR, D), lambda i: (i, 0)),
            pl.BlockSpec((_BR, D), lambda i: (i, 0)),
            pl.BlockSpec((1, D3), lambda i: (0, 0)),
        ],
        out_shape=[
            jax.ShapeDtypeStruct((N, D3), jnp.float32),
            jax.ShapeDtypeStruct((N, D3), jnp.float32),
            jax.ShapeDtypeStruct((N, D3), jnp.float32),
            jax.ShapeDtypeStruct((N, D), jnp.float32),
            jax.ShapeDtypeStruct((N, D), jnp.float32),
            jax.ShapeDtypeStruct((N, D), jnp.float32),
            jax.ShapeDtypeStruct((1, D3), jnp.float32),
        ],
        scratch_shapes=[pltpu.VMEM((1, D3), jnp.float32)],
    )(x, wt, wm, wb, cnt_part)


_C2 = 64
_NF2 = EW // _C2 - 1
_T2 = 16


def _k2_chunk(C, with_inv, b, ib, rb, zo, vb, ivo, acc_v,
              q_h, i_h, v_h, z_h, iv_h, sem):
    full = C == _C2
    for q in range(3):
        pltpu.sync_copy(i_h[q].at[pl.ds(b, C)], ib[q])
    ds = [pltpu.async_copy(q_h[q].at[ib[q]],
                           rb[q] if full else rb[q].at[pl.ds(0, C)], sem)
          for q in range(3)]
    for d in ds:
        d.wait()
    for j in range(D3 // 16):
        k, jl = j // 8, j % 8
        zok = zo[k]

        def row(r, a):
            z = (rb[0][r, pl.ds(j * 16, 16)] + rb[1][r, pl.ds(j * 16, 16)]
                 + rb[2][r, pl.ds(j * 16, 16)])
            zok[r, pl.ds(jl * 16, 16)] = z
            return a + z * z
        acc_v[pl.ds(j * 16, 16)] = lax.fori_loop(
            0, C, row, acc_v[pl.ds(j * 16, 16)])
    for k in range(3):
        pltpu.sync_copy(zo[k] if full else zo[k].at[pl.ds(0, C)],
                        z_h.at[k, pl.ds(b, C)])
    if with_inv:
        for k in range(3):
            pltpu.async_copy(v_h[k].at[ib[k]],
                             vb if full else vb.at[pl.ds(0, C)], sem).wait()

            def ivrow(r, _):
                ivo[r, pl.ds(k * 16, 16)] = vb[r, pl.ds(0, 16)]
                return 0
            lax.fori_loop(0, C, ivrow, 0)
        pltpu.sync_copy(ivo if full else ivo.at[pl.ds(0, C)],
                        iv_h.at[pl.ds(b, C)])


def _make_k2(with_inv):
    def body(q0_h, q1_h, q2_h, i0_h, i1_h, i2_h, v0_h, v1_h, v2_h,
             *outs, ib0, ib1, ib2, it0, it1, it2,
             rb0, rb1, rb2, zo0, zo1, zo2, vb, ivo, acc_v, sem):
        if with_inv:
            ssq_h, z_h, iv_h = outs
        else:
            ssq_h, z_h = outs
            iv_h = None
        _, _, wid = _worker()
        for j in range(D3 // 16):
            acc_v[pl.ds(j * 16, 16)] = jnp.zeros((16,), jnp.float32)
        base0 = wid * EW
        ib = (ib0, ib1, ib2)
        it = (it0, it1, it2)
        rb = (rb0, rb1, rb2)
        zo = (zo0, zo1, zo2)
        q_h = (q0_h, q1_h, q2_h)
        i_h = (i0_h, i1_h, i2_h)
        v_h = (v0_h, v1_h, v2_h)

        def chunk(t, _):
            _k2_chunk(_C2, with_inv, base0 + t * _C2, ib, rb, zo, vb, ivo,
                      acc_v, q_h, i_h, v_h, z_h, iv_h, sem)
            return 0
        lax.fori_loop(0, _NF2 + 1, chunk, 0)
        _k2_chunk(_T2, with_inv, base0 + (_NF2 + 1) * _C2, it, rb, zo, vb,
                  ivo, acc_v, q_h, i_h, v_h, z_h, iv_h, sem)
        pltpu.sync_copy(acc_v, ssq_h.at[wid])

    out_type = [
        jax.ShapeDtypeStruct((NW, D3), jnp.float32),
        jax.ShapeDtypeStruct((3, E, D), jnp.float32),
    ]
    if with_inv:
        out_type.append(jax.ShapeDtypeStruct((E, D), jnp.float32))
    return pl.kernel(
        body,
        out_type=out_type,
        mesh=_MESH,
        scratch_types=dict(
            ib0=pltpu.VMEM((_C2,), jnp.int32),
            ib1=pltpu.VMEM((_C2,), jnp.int32),
            ib2=pltpu.VMEM((_C2,), jnp.int32),
            it0=pltpu.VMEM((_T2,), jnp.int32),
            it1=pltpu.VMEM((_T2,), jnp.int32),
            it2=pltpu.VMEM((_T2,), jnp.int32),
            rb0=pltpu.VMEM((_C2, D3), jnp.float32),
            rb1=pltpu.VMEM((_C2, D3), jnp.float32),
            rb2=pltpu.VMEM((_C2, D3), jnp.float32),
            zo0=pltpu.VMEM((_C2, D), jnp.float32),
            zo1=pltpu.VMEM((_C2, D), jnp.float32),
            zo2=pltpu.VMEM((_C2, D), jnp.float32),
            vb=pltpu.VMEM((_C2, D), jnp.float32),
            ivo=pltpu.VMEM((_C2, D), jnp.float32),
            acc_v=pltpu.VMEM((D3,), jnp.float32),
            sem=pltpu.SemaphoreType.DMA,
        ),
        name="sc_stats_inv" if with_inv else "sc_stats",
    )


_k2_inv_call = _make_k2(True)
_k2_call = _make_k2(False)


_C3 = 80
_NF3 = EW // _C3


def _k3_body(z_h, i0_h, i1_h, i2_h, iv_h, sc_h, sh_h, out_h,
             ib0, ib1, ib2, zb0, zb1, zb2, ivb, sc_v, sh_v, acc_s, sem):
    cid, sid, wid = _worker()
    row0 = sid * RPA
    pltpu.sync_copy(sc_h, sc_v)
    pltpu.sync_copy(sh_h, sh_v)

    def zrow(r, _):
        for j8 in range(8):
            zb0[r, pl.ds(j8 * 16, 16)] = jnp.zeros((16,), jnp.float32)
        return 0
    lax.fori_loop(0, _C3, zrow, 0)

    def zchunk(t, _):
        pltpu.sync_copy(zb0, acc_s.at[pl.ds(row0 + t * _C3, _C3)])
        return 0
    lax.fori_loop(0, 7, zchunk, 0)
    pltpu.sync_copy(zb0.at[pl.ds(0, 64)],
                    acc_s.at[pl.ds(row0 + 7 * _C3, 64)])

    @pl.when(sid == NS - 1)
    def _():
        pltpu.sync_copy(zb0.at[pl.ds(0, 16)], acc_s.at[pl.ds(NS * RPA, 16)])
    plsc.subcore_barrier()

    base0 = wid * EW
    ib = (ib0, ib1, ib2)
    zb = (zb0, zb1, zb2)
    i_h = (i0_h, i1_h, i2_h)

    def chunk(t, _):
        b = base0 + t * _C3
        for q in range(3):
            pltpu.sync_copy(i_h[q].at[pl.ds(b, _C3)], ib[q])
        dz = [pltpu.async_copy(z_h.at[q, pl.ds(b, _C3)], zb[q], sem)
              for q in range(3)]
        dv = pltpu.async_copy(iv_h.at[pl.ds(b, _C3)], ivb, sem)
        for d in dz:
            d.wait()
        dv.wait()
        for k in range(3):
            zbk = zb[k]
            for jl in range(8):
                sc = sc_v[k, pl.ds(jl * 16, 16)]
                sh = sh_v[k, pl.ds(jl * 16, 16)]

                def row(r, _):
                    z = zbk[r, pl.ds(jl * 16, 16)]
                    iv = ivb[r, pl.ds(k * 16, 16)]
                    zbk[r, pl.ds(jl * 16, 16)] = (
                        jnp.maximum(z * sc + sh, 0.0) * iv)
                    return 0
                lax.fori_loop(0, _C3, row, 0)
            pltpu.sync_copy(zbk, acc_s.at[ib[k]], add=True)
        return 0
    lax.fori_loop(0, _NF3, chunk, 0)
    plsc.subcore_barrier()
    pltpu.sync_copy(acc_s.at[pl.ds(row0, RPA)],
                    out_h.at[cid, pl.ds(row0, RPA)])

    @pl.when(sid == NS - 1)
    def _():
        pltpu.sync_copy(acc_s.at[pl.ds(NS * RPA, 16)],
                        out_h.at[cid, pl.ds(NS * RPA, 16)])


_k3_call = pl.kernel(
    _k3_body,
    out_type=jax.ShapeDtypeStruct((NC, N, D), jnp.float32),
    mesh=_MESH,
    scratch_types=dict(
        ib0=pltpu.VMEM((_C3,), jnp.int32),
        ib1=pltpu.VMEM((_C3,), jnp.int32),
        ib2=pltpu.VMEM((_C3,), jnp.int32),
        zb0=pltpu.VMEM((_C3, D), jnp.float32),
        zb1=pltpu.VMEM((_C3, D), jnp.float32),
        zb2=pltpu.VMEM((_C3, D), jnp.float32),
        ivb=pltpu.VMEM((_C3, D), jnp.float32),
        sc_v=pltpu.VMEM((3, D), jnp.float32),
        sh_v=pltpu.VMEM((3, D), jnp.float32),
        acc_s=pltpu.VMEM_SHARED((N, D), jnp.float32),
        sem=pltpu.SemaphoreType.DMA,
    ),
    name="sc_scatter",
)


def _k4_body(mp_ref, x_ref, w_ref, b_ref, g_ref, be_ref, o_ref):
    m = mp_ref[0] + mp_ref[1]
    h = lax.dot_general(m, w_ref[...], (((1,), (1,)), ((), ())),
                        preferred_element_type=jnp.float32) + b_ref[...]
    mean = jnp.mean(h, axis=0, keepdims=True)
    var = jnp.mean((h - mean) ** 2, axis=0, keepdims=True)
    y = g_ref[...] * (h - mean) / jnp.sqrt(var + EPS) + be_ref[...]
    o_ref[...] = x_ref[...] + jnp.maximum(y, 0.0)


def _k4_call(m_part, x, w, b, g, be):
    return pl.pallas_call(
        _k4_body,
        out_shape=jax.ShapeDtypeStruct((N, D), jnp.float32),
    )(m_part, x, w, b.reshape(1, D), g.reshape(1, D), be.reshape(1, D))


def kernel(x, term_walk_index, W_T, b_T, g_T, be_T, W_M, b_M, g_M, be_M,
           W_B, b_B, g_B, be_B, W_TW, b_TW, g_TW, be_TW):
    idx0 = term_walk_index[0]
    idx1 = term_walk_index[1]
    idx2 = term_walk_index[2]
    cnt_part = _counts_call(idx0, idx1, idx2)

    iv48 = None
    for i in range(2):
        q0, q1, q2, v0, v1, v2, sumz = _k1_call(
            x, W_T[i], W_M[i], W_B[i], cnt_part)
        if i == 0:
            ssq_part, zplanes, iv48 = _k2_inv_call(
                q0, q1, q2, idx0, idx1, idx2, v0, v1, v2)
        else:
            ssq_part, zplanes = _k2_call(
                q0, q1, q2, idx0, idx1, idx2, v0, v1, v2)
        ssq = jnp.sum(ssq_part, axis=0)
        mz = sumz.reshape(D3) / E
        var = ssq / E - mz * mz
        g_cat = jnp.concatenate([g_T[i], g_M[i], g_B[i]])
        be_cat = jnp.concatenate([be_T[i], be_M[i], be_B[i]])
        scale = g_cat / jnp.sqrt(var + EPS)
        shift = be_cat - mz * scale
        m_part = _k3_call(zplanes, idx0, idx1, idx2, iv48,
                          scale.reshape(3, D), shift.reshape(3, D))
        x = _k4_call(m_part, x, W_TW[i], b_TW[i], g_TW[i], be_TW[i])
    return x
